# Initial kernel scaffold; baseline (speedup 1.0000x reference)
import jax
import jax.numpy as jnp
from jax.experimental import pallas as pl


def kernel(hidden_states, gate_w, gate_up_w, down_w):
    raise NotImplementedError("write your pallas kernel here")


# speedup vs baseline: 1.8032x; 1.8032x over previous
import functools

import jax
import jax.numpy as jnp
from jax import lax
from jax.experimental import pallas as pl
from jax.experimental.pallas import tpu as pltpu
from jax.experimental.pallas import tpu_sc as plsc

K = 2
BT = 256
BFF = 512
RT = 256
NC, NS = 2, 16
NW = NC * NS


def _router_body(x_ref, gw_ref, sel_ref, wts_ref):
    x = x_ref[...]
    logits = lax.dot_general(x, gw_ref[...], (((1,), (1,)), ((), ())),
                             preferred_element_type=jnp.float32)
    e = logits.shape[1]
    ii = lax.broadcasted_iota(jnp.int32, logits.shape, 1)
    m1 = jnp.max(logits, axis=1, keepdims=True)
    a1 = jnp.min(jnp.where(logits >= m1, ii, e), axis=1, keepdims=True)
    masked = jnp.where(ii == a1, -jnp.inf, logits)
    m2 = jnp.max(masked, axis=1, keepdims=True)
    a2 = jnp.min(jnp.where(masked >= m2, ii, e), axis=1, keepdims=True)
    w1 = 1.0 / (1.0 + jnp.exp(m2 - m1))
    sel_ref[...] = jnp.concatenate([a1, a2], axis=1)
    wts_ref[...] = jnp.concatenate([w1, 1.0 - w1], axis=1)


def _route(x, gate_w):
    t, d = x.shape
    e = gate_w.shape[0]
    sel, wts = pl.pallas_call(
        _router_body,
        grid=(t // RT,),
        in_specs=[
            pl.BlockSpec((RT, d), lambda i: (i, 0)),
            pl.BlockSpec((e, d), lambda i: (0, 0)),
        ],
        out_specs=[
            pl.BlockSpec((RT, K), lambda i: (i, 0)),
            pl.BlockSpec((RT, K), lambda i: (i, 0)),
        ],
        out_shape=[
            jax.ShapeDtypeStruct((t, K), jnp.int32),
            jax.ShapeDtypeStruct((t, K), jnp.float32),
        ],
    )(x, gate_w)
    return sel, wts


def _dispatch_sc(x, src_tok, p):
    t, d = x.shape
    rows_per_w = p // NW
    ch = min(rows_per_w, 64)
    assert rows_per_w % ch == 0 and (ch * d * 4) <= 400_000
    mesh = plsc.VectorSubcoreMesh(core_axis_name="c", subcore_axis_name="s", num_cores=NC, num_subcores=NS)

    @functools.partial(
        pl.kernel,
        out_type=jax.ShapeDtypeStruct((p, d), x.dtype),
        mesh=mesh,
        scratch_types=[
            pltpu.VMEM((ch,), jnp.int32),
            pltpu.VMEM((ch, d), x.dtype),
            pltpu.SemaphoreType.DMA,
        ],
    )
    def dispatch(x_hbm, idx_hbm, xs_hbm, idx_v, rows_v, sem):
        wid = lax.axis_index("s") * NC + lax.axis_index("c")
        base = wid * rows_per_w
        for c in range(rows_per_w // ch):
            off = base + c * ch
            pltpu.sync_copy(idx_hbm.at[pl.ds(off, ch)], idx_v)
            pltpu.async_copy(x_hbm.at[idx_v], rows_v, sem).wait()
            pltpu.sync_copy(rows_v, xs_hbm.at[pl.ds(off, ch)])

    return dispatch(x, src_tok)


def _combine_sc(os_rows, pos0, pos1):
    t = pos0.shape[0]
    d = os_rows.shape[1]
    rows_per_w = t // NW
    ch = min(rows_per_w, 32)
    assert rows_per_w % ch == 0
    mesh = plsc.VectorSubcoreMesh(core_axis_name="c", subcore_axis_name="s", num_cores=NC, num_subcores=NS)

    @functools.partial(
        pl.kernel,
        out_type=jax.ShapeDtypeStruct((t, d), jnp.float32),
        mesh=mesh,
        scratch_types=[
            pltpu.VMEM((ch,), jnp.int32),
            pltpu.VMEM((ch,), jnp.int32),
            pltpu.VMEM((ch, d), jnp.float32),
            pltpu.VMEM((ch, d), jnp.float32),
            pltpu.SemaphoreType.DMA,
            pltpu.SemaphoreType.DMA,
        ],
    )
    def combine(os_hbm, p0_hbm, p1_hbm, out_hbm, i0_v, i1_v, a_v, b_v, s0, s1):
        wid = lax.axis_index("s") * NC + lax.axis_index("c")
        base = wid * rows_per_w
        nch = d // 16
        for c in range(rows_per_w // ch):
            off = base + c * ch
            pltpu.sync_copy(p0_hbm.at[pl.ds(off, ch)], i0_v)
            pltpu.sync_copy(p1_hbm.at[pl.ds(off, ch)], i1_v)
            cp0 = pltpu.async_copy(os_hbm.at[i0_v], a_v, s0)
            cp1 = pltpu.async_copy(os_hbm.at[i1_v], b_v, s1)
            cp0.wait()
            cp1.wait()
            for r in range(ch):
                def add_body(i, _, r=r):
                    col = i * 64
                    for u in range(4):
                        sl = pl.ds(col + 16 * u, 16)
                        a_v[r, sl] = a_v[r, sl] + b_v[r, sl]
                    return _
                lax.fori_loop(0, nch // 4, add_body, None)
            pltpu.sync_copy(a_v, out_hbm.at[pl.ds(off, ch)])

    return combine(os_rows, pos0, pos1)


def _gelu_exact(x):
    return 0.5 * x * (1.0 + lax.erf(x * 0.7071067811865476))


def _mlp_body(nj, be_ref, xs_ref, g_ref, u_ref, d_ref, wv_ref, os_ref):
    j = pl.program_id(1)
    xs = xs_ref[...]
    g = lax.dot_general(xs, g_ref[0], (((1,), (1,)), ((), ())),
                        preferred_element_type=jnp.float32)
    u = lax.dot_general(xs, u_ref[0], (((1,), (1,)), ((), ())),
                        preferred_element_type=jnp.float32)
    h = u * _gelu_exact(g)
    part = lax.dot_general(h, d_ref[0], (((1,), (1,)), ((), ())),
                           preferred_element_type=jnp.float32)

    @pl.when(j == 0)
    def _():
        os_ref[...] = part

    @pl.when(j > 0)
    def _():
        os_ref[...] = os_ref[...] + part

    @pl.when(j == nj - 1)
    def _():
        os_ref[...] = os_ref[...] * wv_ref[...]


def _grouped_mlp(xs, gate_up_w, down_w, be, wv_col):
    p, d = xs.shape
    e, ff2, _ = gate_up_w.shape
    ff = ff2 // 2
    nr, nj = p // BT, ff // BFF
    grid_spec = pltpu.PrefetchScalarGridSpec(
        num_scalar_prefetch=1,
        grid=(nr, nj),
        in_specs=[
            pl.BlockSpec((BT, d), lambda i, j, be: (i, 0)),
            pl.BlockSpec((1, BFF, d), lambda i, j, be: (be[i], j, 0)),
            pl.BlockSpec((1, BFF, d), lambda i, j, be: (be[i], nj + j, 0)),
            pl.BlockSpec((1, d, BFF), lambda i, j, be: (be[i], 0, j)),
            pl.BlockSpec((BT, 1), lambda i, j, be: (i, 0)),
        ],
        out_specs=pl.BlockSpec((BT, d), lambda i, j, be: (i, 0)),
    )
    return pl.pallas_call(
        functools.partial(_mlp_body, nj),
        grid_spec=grid_spec,
        out_shape=jax.ShapeDtypeStruct((p, d), jnp.float32),
        compiler_params=pltpu.CompilerParams(
            dimension_semantics=("arbitrary", "arbitrary")),
    )(be, xs, gate_up_w, gate_up_w, down_w, wv_col)


def _plan(sel, wts, e, p):
    tk = sel.size
    eflat = sel.reshape(-1)
    kord = jnp.argsort(eflat, stable=True).astype(jnp.int32)
    sorted_e = eflat[kord]
    counts = jnp.bincount(eflat, length=e)
    padded = ((counts + BT - 1) // BT) * BT
    pend = jnp.cumsum(padded)
    poff = pend - padded
    starts = jnp.cumsum(counts) - counts
    ar = jnp.arange(tk, dtype=jnp.int32)
    rank = ar - starts[sorted_e].astype(jnp.int32)
    dst = (poff[sorted_e].astype(jnp.int32) + rank)
    src_tok = jnp.zeros((p,), jnp.int32).at[dst].set(kord // K)
    wv = jnp.zeros((p,), jnp.float32).at[dst].set(wts.reshape(-1)[kord])
    tile_ends = pend // BT
    nr = p // BT
    tiles = jnp.arange(nr, dtype=jnp.int32)
    be = jnp.sum(tiles[:, None] >= tile_ends[None, :].astype(jnp.int32),
                 axis=1).astype(jnp.int32)
    be = jnp.minimum(be, e - 1)
    pos = jnp.zeros((tk,), jnp.int32).at[kord].set(dst).reshape(-1, K)
    return src_tok, wv, be, pos[:, 0], pos[:, 1]


def kernel(hidden_states, gate_w, gate_up_w, down_w):
    b, s, d = hidden_states.shape
    e = gate_w.shape[0]
    t = b * s
    p = t * K + e * BT
    assert t % RT == 0 and p % BT == 0 and p % (8 * NW) == 0

    x = hidden_states.reshape(t, d)
    sel, wts = _route(x, gate_w)
    src_tok, wv, be, pos0, pos1 = _plan(sel, wts, e, p)
    xs = _dispatch_sc(x, src_tok, p)
    os_rows = _grouped_mlp(xs, gate_up_w, down_w, be, wv.reshape(p, 1))
    out = _combine_sc(os_rows, pos0, pos1)
    return out.reshape(b, s, d)
